# baseline (speedup 1.0000x reference)
import jax
import jax.numpy as jnp
import numpy as np
from jax.experimental import pallas as pl
from jax.experimental.pallas import tpu as pltpu

ROWS = 32
NCOLS = 1_000_000
BLK = 16384
CH = 256
NCH = BLK // CH
GRID = (NCOLS + BLK - 1) // BLK

_TINY = np.float32(np.finfo(np.float32).tiny)
_KS = (np.uint32(0), np.uint32(42), np.uint32(0x1BD11BDA ^ 42))
_ROTS = ((13, 15, 26, 6), (17, 29, 16, 24), (13, 15, 26, 6),
         (17, 29, 16, 24), (13, 15, 26, 6))


def _threefry_bits(flat):
    x0 = jnp.zeros_like(flat)
    x1 = flat + _KS[1]
    for g in range(5):
        for r in _ROTS[g]:
            x0 = x0 + x1
            x1 = ((x1 << r) | (x1 >> (32 - r))) ^ x0
        x0 = x0 + _KS[(g + 1) % 3]
        x1 = x1 + _KS[(g + 2) % 3] + np.uint32(g + 1)
    return x0 ^ x1


def _gumbel(bits):
    fb = (bits >> 9) | np.uint32(0x3F800000)
    u0 = jax.lax.bitcast_convert_type(fb, jnp.float32) - np.float32(1.0)
    u = jnp.maximum(u0, _TINY)
    return -jnp.log(-jnp.log(u))


def _sample_kernel(x_ref, out_ref, vacc_ref, cacc_ref):
    k = pl.program_id(0)

    @pl.when(k == 0)
    def _init():
        vacc_ref[...] = jnp.full((ROWS, CH), -jnp.inf, jnp.float32)
        cacc_ref[...] = jnp.zeros((ROWS, CH), jnp.int32)

    lane = jax.lax.broadcasted_iota(jnp.int32, (ROWS, CH), 1)
    rowmul = jax.lax.broadcasted_iota(jnp.int32, (ROWS, CH), 0) * NCOLS
    base_flat = rowmul + lane

    def process(masked):
        for c in range(NCH):
            off = k * BLK + c * CH
            flat = (base_flat + off).astype(jnp.uint32)
            g = _gumbel(_threefry_bits(flat))
            v = x_ref[:, c * CH:(c + 1) * CH] + g
            av = vacc_ref[...]
            pred = v > av
            if masked:
                pred = pred & (lane < NCOLS - off)
            vacc_ref[...] = jnp.where(pred, v, av)
            cacc_ref[...] = jnp.where(pred, flat.astype(jnp.int32),
                                      cacc_ref[...])

    @pl.when(k < GRID - 1)
    def _main():
        process(False)

    @pl.when(k == GRID - 1)
    def _tail():
        process(True)
        av = vacc_ref[...]
        m = jnp.max(av, axis=1, keepdims=True)
        col = cacc_ref[...] - rowmul
        out_ref[...] = jnp.min(jnp.where(av == m, col, jnp.int32(2**30)),
                               axis=1, keepdims=True)


def kernel(outputs):
    return pl.pallas_call(
        _sample_kernel,
        grid=(GRID,),
        in_specs=[pl.BlockSpec((ROWS, BLK), lambda k: (0, k))],
        out_specs=pl.BlockSpec((ROWS, 1), lambda k: (0, 0)),
        out_shape=jax.ShapeDtypeStruct((ROWS, 1), jnp.int32),
        scratch_shapes=[pltpu.VMEM((ROWS, CH), jnp.float32),
                        pltpu.VMEM((ROWS, CH), jnp.int32)],
        compiler_params=pltpu.CompilerParams(
            dimension_semantics=("arbitrary",)),
    )(outputs)


# speedup vs baseline: 1.0164x; 1.0164x over previous
import jax
import jax.numpy as jnp
import numpy as np
from jax.experimental import pallas as pl
from jax.experimental.pallas import tpu as pltpu

ROWS = 32
NCOLS = 1_000_000
BLK = 16384
CH = 256
NCH = BLK // CH
GRID = (NCOLS + BLK - 1) // BLK

_TINY = np.float32(np.finfo(np.float32).tiny)
_KS = (np.uint32(0), np.uint32(42), np.uint32(0x1BD11BDA ^ 42))
_ROTS = ((13, 15, 26, 6), (17, 29, 16, 24), (13, 15, 26, 6),
         (17, 29, 16, 24), (13, 15, 26, 6))


def _threefry_bits(x1):
    x0 = jnp.zeros_like(x1)
    for g in range(5):
        for r in _ROTS[g]:
            x0 = x0 + x1
            x1 = ((x1 << r) | (x1 >> (32 - r))) ^ x0
        x0 = x0 + _KS[(g + 1) % 3]
        x1 = x1 + _KS[(g + 2) % 3] + np.uint32(g + 1)
    return x0 ^ x1


def _x_plus_gumbel(x, bits):
    fb = (bits >> 9) | np.uint32(0x3F800000)
    u0 = jax.lax.bitcast_convert_type(fb, jnp.float32) - np.float32(1.0)
    u = jnp.maximum(u0, _TINY)
    return x - jnp.log(-jnp.log(u))


def _sample_kernel(x_ref, out_ref, vacc_ref, cacc_ref):
    k = pl.program_id(0)

    @pl.when(k == 0)
    def _init():
        vacc_ref[...] = jnp.full((ROWS, CH), -jnp.inf, jnp.float32)
        cacc_ref[...] = jnp.zeros((ROWS, CH), jnp.int32)

    lane = jax.lax.broadcasted_iota(jnp.int32, (ROWS, CH), 1)
    rowmul = jax.lax.broadcasted_iota(jnp.int32, (ROWS, CH), 0) * NCOLS
    base_flat = rowmul + lane

    def process(masked):
        for c in range(NCH):
            off = k * BLK + c * CH
            flat42 = (base_flat + (off + 42)).astype(jnp.uint32)
            v = _x_plus_gumbel(x_ref[:, c * CH:(c + 1) * CH],
                               _threefry_bits(flat42))
            av = vacc_ref[...]
            pred = v > av
            if masked:
                pred = pred & (lane < NCOLS - off)
            vacc_ref[...] = jnp.where(pred, v, av)
            cacc_ref[...] = jnp.where(pred, flat42.astype(jnp.int32),
                                      cacc_ref[...])

    @pl.when(k < GRID - 1)
    def _main():
        process(False)

    @pl.when(k == GRID - 1)
    def _tail():
        process(True)
        av = vacc_ref[...]
        m = jnp.max(av, axis=1, keepdims=True)
        col = cacc_ref[...] - rowmul - 42
        out_ref[...] = jnp.min(jnp.where(av == m, col, jnp.int32(2**30)),
                               axis=1, keepdims=True)


def kernel(outputs):
    return pl.pallas_call(
        _sample_kernel,
        grid=(GRID,),
        in_specs=[pl.BlockSpec((ROWS, BLK), lambda k: (0, k))],
        out_specs=pl.BlockSpec((ROWS, 1), lambda k: (0, 0)),
        out_shape=jax.ShapeDtypeStruct((ROWS, 1), jnp.int32),
        scratch_shapes=[pltpu.VMEM((ROWS, CH), jnp.float32),
                        pltpu.VMEM((ROWS, CH), jnp.int32)],
        compiler_params=pltpu.CompilerParams(
            dimension_semantics=("arbitrary",)),
    )(outputs)
